# baseline (device time: 16733 ns/iter reference)
import jax
import jax.numpy as jnp
from jax import lax
from jax.experimental import pallas as pl
from jax.experimental.pallas import tpu as pltpu

N_DEV = 8
CAP = 102
LANES = 128


def kernel(x, router_W, route_idx, expert_W):
    del router_W
    n_tok, d = x.shape
    n_exp_loc, _, h = expert_W.shape

    def body(x_ref, route_ref, w_ref, out_ref,
             w_all, cnt_all, w_src, copy_sem,
             w_send, w_recv, c_send, c_recv):
        my = lax.axis_index("i")

        bar = pltpu.get_barrier_semaphore()
        for k in range(1, N_DEV):
            pl.semaphore_signal(
                bar, inc=1,
                device_id=(lax.rem(my + k, N_DEV),),
                device_id_type=pl.DeviceIdType.MESH,
            )
        pl.semaphore_wait(bar, N_DEV - 1)

        routes = route_ref[...]
        e_ids = lax.broadcasted_iota(jnp.int32, (n_tok, LANES), 1)
        onehot = (routes == e_ids).astype(jnp.int32)
        counts = jnp.sum(onehot, axis=0, keepdims=True)
        cnt_all[pl.ds(my, 1), :, :] = counts.reshape(1, 1, LANES)

        w_src[...] = w_ref[...].astype(jnp.bfloat16)
        cp = pltpu.make_async_copy(w_src, w_all.at[my], copy_sem)
        cp.start()

        rdmas = []
        for k in range(1, N_DEV):
            dst = lax.rem(my + k, N_DEV)
            w_rd = pltpu.make_async_remote_copy(
                src_ref=w_src, dst_ref=w_all.at[my],
                send_sem=w_send.at[k], recv_sem=w_recv.at[my],
                device_id=(dst,), device_id_type=pl.DeviceIdType.MESH,
            )
            c_rd = pltpu.make_async_remote_copy(
                src_ref=cnt_all.at[my], dst_ref=cnt_all.at[my],
                send_sem=c_send.at[k], recv_sem=c_recv.at[my],
                device_id=(dst,), device_id_type=pl.DeviceIdType.MESH,
            )
            c_rd.start()
            w_rd.start()
            rdmas.append((w_rd, c_rd))

        for dd in range(N_DEV):
            @pl.when(dd != my)
            def _():
                pltpu.make_async_remote_copy(
                    src_ref=cnt_all.at[dd], dst_ref=cnt_all.at[dd],
                    send_sem=c_send.at[0], recv_sem=c_recv.at[dd],
                    device_id=(my,), device_id_type=pl.DeviceIdType.MESH,
                ).wait_recv()

        cnt = cnt_all[...].reshape(N_DEV, LANES)
        shard_row = lax.broadcasted_iota(jnp.int32, (N_DEV, LANES), 0)
        prefix = jnp.sum(
            jnp.where(shard_row < my, cnt, 0), axis=0, keepdims=True
        )
        row = lax.broadcasted_iota(jnp.int32, (n_tok, n_tok), 0)
        col = lax.broadcasted_iota(jnp.int32, (n_tok, n_tok), 1)
        tri = (col <= row).astype(jnp.bfloat16)
        csum = jnp.dot(
            tri, onehot.astype(jnp.bfloat16),
            preferred_element_type=jnp.float32,
        )
        kept = onehot * (
            (prefix.astype(jnp.float32) + csum) <= CAP
        ).astype(jnp.int32)
        kept_bf = kept.astype(jnp.bfloat16)
        xv = x_ref[...].astype(jnp.bfloat16)

        acc = jnp.zeros((n_tok, h), jnp.float32)
        cp.wait()
        for dd in range(N_DEV):
            @pl.when(dd != my)
            def _():
                pltpu.make_async_remote_copy(
                    src_ref=w_all.at[dd], dst_ref=w_all.at[dd],
                    send_sem=w_send.at[0], recv_sem=w_recv.at[dd],
                    device_id=(my,), device_id_type=pl.DeviceIdType.MESH,
                ).wait_recv()
            for s in range(n_exp_loc):
                e = dd * n_exp_loc + s
                m = kept_bf[:, e:e + 1]
                acc = acc + jnp.dot(
                    xv * m, w_all[dd, s],
                    preferred_element_type=jnp.float32,
                )
        out_ref[...] = acc

        for w_rd, c_rd in rdmas:
            w_rd.wait_send()
            c_rd.wait_send()

    return pl.pallas_call(
        body,
        out_shape=jax.ShapeDtypeStruct((n_tok, h), jnp.float32),
        in_specs=[pl.BlockSpec(memory_space=pltpu.VMEM)] * 3,
        out_specs=pl.BlockSpec(memory_space=pltpu.VMEM),
        scratch_shapes=[
            pltpu.VMEM((N_DEV, n_exp_loc, d, h), jnp.bfloat16),
            pltpu.VMEM((N_DEV, 1, LANES), jnp.int32),
            pltpu.VMEM((n_exp_loc, d, h), jnp.bfloat16),
            pltpu.SemaphoreType.DMA,
            pltpu.SemaphoreType.DMA((N_DEV,)),
            pltpu.SemaphoreType.DMA((N_DEV,)),
            pltpu.SemaphoreType.DMA((N_DEV,)),
            pltpu.SemaphoreType.DMA((N_DEV,)),
        ],
        compiler_params=pltpu.CompilerParams(collective_id=0),
    )(x, route_idx, expert_W)


# device time: 14455 ns/iter; 1.1576x vs baseline; 1.1576x over previous
import jax
import jax.numpy as jnp
from jax import lax
from jax.experimental import pallas as pl
from jax.experimental.pallas import tpu as pltpu

N_DEV = 8
CAP = 102
LANES = 128
SCALE_LANE = 16


def kernel(x, router_W, route_idx, expert_W):
    del router_W
    n_tok, d = x.shape
    n_exp_loc, _, h = expert_W.shape

    def body(x_ref, route_ref, w_ref, out_ref,
             w_all, cnt_all, w_src, copy_sem,
             w_send, w_recv, c_send, c_recv):
        my = lax.axis_index("i")

        routes = route_ref[...]
        e_ids = lax.broadcasted_iota(jnp.int32, (n_tok, LANES), 1)
        onehot = (routes == e_ids).astype(jnp.int32)
        counts = jnp.sum(onehot.astype(jnp.float32), axis=0,
                         keepdims=True)

        chunk = w_ref[...]
        amax = jnp.max(jnp.abs(chunk))
        scale = amax / 127.0
        w_src[...] = jnp.clip(
            jnp.round(chunk * (127.0 / amax)), -127.0, 127.0
        ).astype(jnp.int8)
        lane = lax.broadcasted_iota(jnp.int32, (1, LANES), 1)
        cnt_all[pl.ds(my, 1), :, :] = (
            counts + jnp.where(lane == SCALE_LANE, scale, 0.0)
        ).reshape(1, 1, LANES)

        cp = pltpu.make_async_copy(w_src, w_all.at[my], copy_sem)
        cp.start()

        bar = pltpu.get_barrier_semaphore()
        for k in range(1, N_DEV):
            pl.semaphore_signal(
                bar, inc=1,
                device_id=(lax.rem(my + k, N_DEV),),
                device_id_type=pl.DeviceIdType.MESH,
            )
        pl.semaphore_wait(bar, N_DEV - 1)

        rdmas = []
        for k in range(1, N_DEV):
            dst = lax.rem(my + k, N_DEV)
            w_rd = pltpu.make_async_remote_copy(
                src_ref=w_src, dst_ref=w_all.at[my],
                send_sem=w_send.at[k], recv_sem=w_recv.at[my],
                device_id=(dst,), device_id_type=pl.DeviceIdType.MESH,
            )
            c_rd = pltpu.make_async_remote_copy(
                src_ref=cnt_all.at[my], dst_ref=cnt_all.at[my],
                send_sem=c_send.at[k], recv_sem=c_recv.at[my],
                device_id=(dst,), device_id_type=pl.DeviceIdType.MESH,
            )
            c_rd.start()
            w_rd.start()
            rdmas.append((w_rd, c_rd))

        for dd in range(N_DEV):
            @pl.when(dd != my)
            def _():
                pltpu.make_async_remote_copy(
                    src_ref=cnt_all.at[dd], dst_ref=cnt_all.at[dd],
                    send_sem=c_send.at[0], recv_sem=c_recv.at[dd],
                    device_id=(my,), device_id_type=pl.DeviceIdType.MESH,
                ).wait_recv()

        cnt = cnt_all[...].reshape(N_DEV, LANES)
        shard_row = lax.broadcasted_iota(jnp.int32, (N_DEV, LANES), 0)
        prefix = jnp.sum(
            jnp.where(shard_row < my, cnt, 0.0), axis=0, keepdims=True
        )
        row = lax.broadcasted_iota(jnp.int32, (n_tok, n_tok), 0)
        col = lax.broadcasted_iota(jnp.int32, (n_tok, n_tok), 1)
        tri = (col <= row).astype(jnp.bfloat16)
        csum = jnp.dot(
            tri, onehot.astype(jnp.bfloat16),
            preferred_element_type=jnp.float32,
        )
        kept = onehot * ((prefix + csum) <= CAP).astype(jnp.int32)
        kept_bf = kept.astype(jnp.bfloat16)
        xv = x_ref[...].astype(jnp.bfloat16)

        acc = jnp.zeros((n_tok, h), jnp.float32)
        cp.wait()
        for dd in range(N_DEV):
            @pl.when(dd != my)
            def _():
                pltpu.make_async_remote_copy(
                    src_ref=w_all.at[dd], dst_ref=w_all.at[dd],
                    send_sem=w_send.at[0], recv_sem=w_recv.at[dd],
                    device_id=(my,), device_id_type=pl.DeviceIdType.MESH,
                ).wait_recv()
            scale_dd = cnt[dd:dd + 1, SCALE_LANE:SCALE_LANE + 1]
            for s in range(n_exp_loc):
                e = dd * n_exp_loc + s
                m = kept_bf[:, e:e + 1] * scale_dd.astype(jnp.bfloat16)
                acc = acc + jnp.dot(
                    xv * m, w_all[dd, s].astype(jnp.bfloat16),
                    preferred_element_type=jnp.float32,
                )
        out_ref[...] = acc

        for w_rd, c_rd in rdmas:
            w_rd.wait_send()
            c_rd.wait_send()

    return pl.pallas_call(
        body,
        out_shape=jax.ShapeDtypeStruct((n_tok, h), jnp.float32),
        in_specs=[pl.BlockSpec(memory_space=pltpu.VMEM)] * 3,
        out_specs=pl.BlockSpec(memory_space=pltpu.VMEM),
        scratch_shapes=[
            pltpu.VMEM((N_DEV, n_exp_loc, d, h), jnp.int8),
            pltpu.VMEM((N_DEV, 1, LANES), jnp.float32),
            pltpu.VMEM((n_exp_loc, d, h), jnp.int8),
            pltpu.SemaphoreType.DMA,
            pltpu.SemaphoreType.DMA((N_DEV,)),
            pltpu.SemaphoreType.DMA((N_DEV,)),
            pltpu.SemaphoreType.DMA((N_DEV,)),
            pltpu.SemaphoreType.DMA((N_DEV,)),
        ],
        compiler_params=pltpu.CompilerParams(collective_id=0),
    )(x, route_idx, expert_W)


# device time: 12796 ns/iter; 1.3077x vs baseline; 1.1296x over previous
import jax
import jax.numpy as jnp
from jax import lax
from jax.experimental import pallas as pl
from jax.experimental.pallas import tpu as pltpu

N_DEV = 8
CAP = 102
LANES = 128
SCALE_LANE = 16


def kernel(x, router_W, route_idx, expert_W):
    del router_W
    n_tok, d = x.shape
    n_exp_loc, _, h = expert_W.shape

    def body(x_ref, route_ref, w_ref, out_ref,
             w_all, cnt_all, w_src, copy_sem,
             w_send, w_recv, c_send, c_recv):
        my = lax.axis_index("i")

        routes = route_ref[...]
        e_ids = lax.broadcasted_iota(jnp.int32, (n_tok, LANES), 1)
        onehot = (routes == e_ids).astype(jnp.int32)
        counts = jnp.sum(onehot.astype(jnp.float32), axis=0,
                         keepdims=True)

        chunk = w_ref[...]
        amax = jnp.max(jnp.abs(chunk))
        scale = amax / 127.0
        w_src[...] = jnp.clip(
            jnp.round(chunk * (127.0 / amax)), -127.0, 127.0
        ).astype(jnp.int8)
        lane = lax.broadcasted_iota(jnp.int32, (1, LANES), 1)
        cnt_all[pl.ds(my, 1), :, :] = (
            counts + jnp.where(lane == SCALE_LANE, scale, 0.0)
        ).reshape(1, 1, LANES)

        cp = pltpu.make_async_copy(w_src, w_all.at[my], copy_sem)
        cp.start()

        bar = pltpu.get_barrier_semaphore()
        for k in range(1, N_DEV):
            pl.semaphore_signal(
                bar, inc=1,
                device_id=(lax.rem(my + k, N_DEV),),
                device_id_type=pl.DeviceIdType.MESH,
            )
        pl.semaphore_wait(bar, N_DEV - 1)

        rdmas = []
        for k in range(1, N_DEV):
            dst = lax.rem(my + k, N_DEV)
            w_rd = pltpu.make_async_remote_copy(
                src_ref=w_src, dst_ref=w_all.at[my],
                send_sem=w_send.at[k], recv_sem=w_recv.at[my],
                device_id=(dst,), device_id_type=pl.DeviceIdType.MESH,
            )
            c_rd = pltpu.make_async_remote_copy(
                src_ref=cnt_all.at[my], dst_ref=cnt_all.at[my],
                send_sem=c_send.at[k], recv_sem=c_recv.at[my],
                device_id=(dst,), device_id_type=pl.DeviceIdType.MESH,
            )
            c_rd.start()
            w_rd.start()
            rdmas.append((w_rd, c_rd))

        for dd in range(N_DEV):
            @pl.when(dd != my)
            def _():
                pltpu.make_async_remote_copy(
                    src_ref=cnt_all.at[dd], dst_ref=cnt_all.at[dd],
                    send_sem=c_send.at[0], recv_sem=c_recv.at[dd],
                    device_id=(my,), device_id_type=pl.DeviceIdType.MESH,
                ).wait_recv()

        cnt = cnt_all[...].reshape(N_DEV, LANES)
        shard_row = lax.broadcasted_iota(jnp.int32, (N_DEV, LANES), 0)
        prefix = jnp.sum(
            jnp.where(shard_row < my, cnt, 0.0), axis=0, keepdims=True
        )
        row = lax.broadcasted_iota(jnp.int32, (n_tok, n_tok), 0)
        col = lax.broadcasted_iota(jnp.int32, (n_tok, n_tok), 1)
        tri = (col <= row).astype(jnp.bfloat16)
        csum = jnp.dot(
            tri, onehot.astype(jnp.bfloat16),
            preferred_element_type=jnp.float32,
        )
        kept = onehot * ((prefix + csum) <= CAP).astype(jnp.int32)

        s8 = cnt[:, SCALE_LANE:SCALE_LANE + 1]
        dev = lax.broadcasted_iota(jnp.int32, (N_DEV, LANES), 0)
        lane2 = lax.broadcasted_iota(jnp.int32, (N_DEV, LANES), 1)
        sel = (lane2 >= n_exp_loc * dev) & (lane2 < n_exp_loc * (dev + 1))
        scale_row = jnp.sum(
            jnp.where(sel, s8, 0.0), axis=0, keepdims=True
        )
        m_all = (
            kept.astype(jnp.float32) * scale_row
        ).astype(jnp.bfloat16)
        xv = x_ref[...].astype(jnp.bfloat16)

        z = jnp.concatenate(
            [xv * m_all[:, e:e + 1] for e in range(N_DEV * n_exp_loc)],
            axis=1,
        )

        cp.wait()
        for dd in range(N_DEV):
            @pl.when(dd != my)
            def _():
                pltpu.make_async_remote_copy(
                    src_ref=w_all.at[dd], dst_ref=w_all.at[dd],
                    send_sem=w_send.at[0], recv_sem=w_recv.at[dd],
                    device_id=(my,), device_id_type=pl.DeviceIdType.MESH,
                ).wait_recv()
        wq = w_all[...].reshape(N_DEV * n_exp_loc * d, h)
        out_ref[...] = jnp.dot(
            z, wq.astype(jnp.bfloat16),
            preferred_element_type=jnp.float32,
        )

        for w_rd, c_rd in rdmas:
            w_rd.wait_send()
            c_rd.wait_send()

    return pl.pallas_call(
        body,
        out_shape=jax.ShapeDtypeStruct((n_tok, h), jnp.float32),
        in_specs=[pl.BlockSpec(memory_space=pltpu.VMEM)] * 3,
        out_specs=pl.BlockSpec(memory_space=pltpu.VMEM),
        scratch_shapes=[
            pltpu.VMEM((N_DEV, n_exp_loc, d, h), jnp.int8),
            pltpu.VMEM((N_DEV, 1, LANES), jnp.float32),
            pltpu.VMEM((n_exp_loc, d, h), jnp.int8),
            pltpu.SemaphoreType.DMA,
            pltpu.SemaphoreType.DMA((N_DEV,)),
            pltpu.SemaphoreType.DMA((N_DEV,)),
            pltpu.SemaphoreType.DMA((N_DEV,)),
            pltpu.SemaphoreType.DMA((N_DEV,)),
        ],
        compiler_params=pltpu.CompilerParams(collective_id=0),
    )(x, route_idx, expert_W)


# device time: 11443 ns/iter; 1.4623x vs baseline; 1.1182x over previous
import jax
import jax.numpy as jnp
from jax import lax
from jax.experimental import pallas as pl
from jax.experimental.pallas import tpu as pltpu

N_DEV = 8
CAP = 102
LANES = 128
SCALE_LANE = 16


def kernel(x, router_W, route_idx, expert_W):
    del router_W
    n_tok, d = x.shape
    n_exp_loc, _, h = expert_W.shape

    def body(x_ref, route_ref, w_ref, out_ref,
             w_all, cnt_all, w_src, copy_sem,
             w_send, w_recv, c_send, c_recv):
        my = lax.axis_index("i")

        routes = route_ref[...]
        e_ids = lax.broadcasted_iota(jnp.int32, (n_tok, LANES), 1)
        onehot = (routes == e_ids).astype(jnp.int32)
        counts = jnp.sum(onehot.astype(jnp.float32), axis=0,
                         keepdims=True)

        chunk = w_ref[...]
        amax = jnp.max(jnp.abs(chunk))
        scale = amax / 127.0
        w_src[...] = jnp.clip(
            jnp.round(chunk * (127.0 / amax)), -127.0, 127.0
        ).astype(jnp.int8)
        lane = lax.broadcasted_iota(jnp.int32, (1, LANES), 1)
        cnt_all[pl.ds(my, 1), :, :] = (
            counts + jnp.where(lane == SCALE_LANE, scale, 0.0)
        ).reshape(1, 1, LANES)

        cp = pltpu.make_async_copy(w_src, w_all.at[my], copy_sem)
        cp.start()

        bar = pltpu.get_barrier_semaphore()
        for k in range(1, N_DEV):
            pl.semaphore_signal(
                bar, inc=1,
                device_id=(lax.rem(my + k, N_DEV),),
                device_id_type=pl.DeviceIdType.MESH,
            )
        pl.semaphore_wait(bar, N_DEV - 1)

        rdmas = []
        for k in range(1, N_DEV):
            dst = lax.rem(my + k, N_DEV)
            w_rd = pltpu.make_async_remote_copy(
                src_ref=w_src, dst_ref=w_all.at[my],
                send_sem=w_send.at[k], recv_sem=w_recv.at[my],
                device_id=(dst,), device_id_type=pl.DeviceIdType.MESH,
            )
            c_rd = pltpu.make_async_remote_copy(
                src_ref=cnt_all.at[my], dst_ref=cnt_all.at[my],
                send_sem=c_send.at[k], recv_sem=c_recv.at[my],
                device_id=(dst,), device_id_type=pl.DeviceIdType.MESH,
            )
            c_rd.start()
            w_rd.start()
            rdmas.append((w_rd, c_rd))

        for dd in range(N_DEV):
            @pl.when(dd != my)
            def _():
                pltpu.make_async_remote_copy(
                    src_ref=cnt_all.at[dd], dst_ref=cnt_all.at[dd],
                    send_sem=c_send.at[0], recv_sem=c_recv.at[dd],
                    device_id=(my,), device_id_type=pl.DeviceIdType.MESH,
                ).wait_recv()

        cnt = cnt_all[...].reshape(N_DEV, LANES)
        shard_row = lax.broadcasted_iota(jnp.int32, (N_DEV, LANES), 0)
        prefix = jnp.sum(
            jnp.where(shard_row < my, cnt, 0.0), axis=0, keepdims=True
        )
        row = lax.broadcasted_iota(jnp.int32, (n_tok, n_tok), 0)
        col = lax.broadcasted_iota(jnp.int32, (n_tok, n_tok), 1)
        tri = (col <= row).astype(jnp.bfloat16)
        csum = jnp.dot(
            tri, onehot.astype(jnp.bfloat16),
            preferred_element_type=jnp.float32,
        )
        kept = onehot * ((prefix + csum) <= CAP).astype(jnp.int32)

        s8 = cnt[:, SCALE_LANE:SCALE_LANE + 1]
        dev = lax.broadcasted_iota(jnp.int32, (N_DEV, LANES), 0)
        lane2 = lax.broadcasted_iota(jnp.int32, (N_DEV, LANES), 1)
        sel = (lane2 >= n_exp_loc * dev) & (lane2 < n_exp_loc * (dev + 1))
        scale_row = jnp.sum(
            jnp.where(sel, s8, 0.0), axis=0, keepdims=True
        )
        m_all = (
            kept.astype(jnp.float32) * scale_row
        ).astype(jnp.bfloat16)
        xv = x_ref[...].astype(jnp.bfloat16)

        z = jnp.concatenate(
            [xv * m_all[:, e:e + 1] for e in range(N_DEV * n_exp_loc)],
            axis=1,
        )

        cp.wait()
        half = N_DEV // 2
        cols = half * n_exp_loc * d
        acc = jnp.zeros((n_tok, h), jnp.float32)
        for g in range(2):
            for dd in range(g * half, (g + 1) * half):
                @pl.when(dd != my)
                def _():
                    pltpu.make_async_remote_copy(
                        src_ref=w_all.at[dd], dst_ref=w_all.at[dd],
                        send_sem=w_send.at[0], recv_sem=w_recv.at[dd],
                        device_id=(my,), device_id_type=pl.DeviceIdType.MESH,
                    ).wait_recv()
            wq = w_all[g * half:(g + 1) * half].reshape(cols, h)
            acc = acc + jnp.dot(
                z[:, g * cols:(g + 1) * cols], wq.astype(jnp.bfloat16),
                preferred_element_type=jnp.float32,
            )
        out_ref[...] = acc

        for w_rd, c_rd in rdmas:
            w_rd.wait_send()
            c_rd.wait_send()

    return pl.pallas_call(
        body,
        out_shape=jax.ShapeDtypeStruct((n_tok, h), jnp.float32),
        in_specs=[pl.BlockSpec(memory_space=pltpu.VMEM)] * 3,
        out_specs=pl.BlockSpec(memory_space=pltpu.VMEM),
        scratch_shapes=[
            pltpu.VMEM((N_DEV, n_exp_loc, d, h), jnp.int8),
            pltpu.VMEM((N_DEV, 1, LANES), jnp.float32),
            pltpu.VMEM((n_exp_loc, d, h), jnp.int8),
            pltpu.SemaphoreType.DMA,
            pltpu.SemaphoreType.DMA((N_DEV,)),
            pltpu.SemaphoreType.DMA((N_DEV,)),
            pltpu.SemaphoreType.DMA((N_DEV,)),
            pltpu.SemaphoreType.DMA((N_DEV,)),
        ],
        compiler_params=pltpu.CompilerParams(collective_id=0),
    )(x, route_idx, expert_W)
